# Initial kernel scaffold; baseline (speedup 1.0000x reference)
import jax
import jax.numpy as jnp
from jax.experimental import pallas as pl


def kernel(outputs, atom_split, W, b):
    raise NotImplementedError("write your pallas kernel here")


# speedup vs baseline: 9.5857x; 9.5857x over previous
import functools

import jax
import jax.numpy as jnp
import numpy as np
from jax import lax
from jax.experimental import pallas as pl
from jax.experimental.pallas import tpu as pltpu
from jax.experimental.pallas import tpu_sc as plsc

N_ATOMS = 100000
N_INPUT = 128
BATCH = 4096
NK = 11

_MEMBERSHIPS = [(-1.645, 0.283), (-1.08, 0.17), (-0.739, 0.134),
                (-0.468, 0.118), (-0.228, 0.114), (0.0, 0.114),
                (0.228, 0.114), (0.468, 0.118), (0.739, 0.134),
                (1.08, 0.17), (1.645, 0.283)]
_MUS = [float(m) for m, _ in _MEMBERSHIPS]
_INV_SIG = [1.0 / float(s) for _, s in _MEMBERSHIPS]

NPAD = 102400
TA = 1024

NCORE = 2
NSUB = 16
PER_W = NPAD // (NCORE * NSUB)
CH = 640
NCHUNK = PER_W // CH
KPC = CH // 128
KPW = PER_W // 128
ZPW = 264
RACC = ZPW * NSUB
OPW = BATCH // NSUB


def _hist_matmul_body(x_ref, w_ref, y_ref):
    x = x_ref[...]
    es = []
    s = None
    for k in range(NK):
        z = (x - _MUS[k]) * _INV_SIG[k]
        e = jnp.exp(-0.5 * z * z)
        es.append(e)
        s = e if s is None else s + e
    r = 1.0 / s
    acc = None
    for k in range(NK):
        p = es[k] * r
        t = jnp.dot(p, w_ref[k], preferred_element_type=jnp.float32)
        acc = t if acc is None else acc + t
    y_ref[...] = acc


def _seg_sum_sc(y_hbm, ids_hbm, zeros_hbm, out_hbm, rows_v, ids_v, acc_sh):
    c = lax.axis_index("c")
    s = lax.axis_index("s")
    w = c * NSUB + s
    pltpu.sync_copy(zeros_hbm.at[pl.ds(s * ZPW, ZPW)], rows_v.at[pl.ds(0, ZPW)])
    pltpu.sync_copy(rows_v.at[pl.ds(0, ZPW)], acc_sh.at[pl.ds(s * ZPW, ZPW)])
    pltpu.sync_copy(ids_hbm.at[w], ids_v)
    plsc.subcore_barrier()
    for t in range(NCHUNK):
        base = w * PER_W + t * CH
        pltpu.sync_copy(y_hbm.at[pl.ds(base, CH)], rows_v)
        for j in range(KPC):
            pltpu.sync_copy(rows_v.at[pl.ds(j * 128, 128)],
                            acc_sh.at[ids_v.at[t * KPC + j]], add=True)
    plsc.subcore_barrier()
    pltpu.sync_copy(acc_sh.at[pl.ds(s * OPW, OPW)], rows_v.at[pl.ds(0, OPW)])
    pltpu.sync_copy(rows_v.at[pl.ds(0, OPW)],
                    out_hbm.at[c, pl.ds(s * OPW, OPW)])


def _finish_body(p_ref, b_ref, o_ref):
    o_ref[...] = jnp.tanh(p_ref[0] + p_ref[1] + b_ref[...])


@jax.jit
def kernel(outputs, atom_split, W, b):
    Wr = W.reshape(N_INPUT, NK, N_INPUT).transpose(1, 0, 2)
    xpad = jnp.pad(outputs, ((0, NPAD - N_ATOMS), (0, 0)))
    ids = jnp.concatenate([
        atom_split.astype(jnp.int32),
        jnp.full((NPAD - N_ATOMS,), BATCH, jnp.int32),
    ])
    ids3d = ids.reshape(NCORE * NSUB, KPW, 128)
    zeros = jnp.zeros((RACC, N_INPUT), jnp.float32)

    y = pl.pallas_call(
        _hist_matmul_body,
        grid=(NPAD // TA,),
        in_specs=[
            pl.BlockSpec((TA, N_INPUT), lambda i: (i, 0)),
            pl.BlockSpec((NK, N_INPUT, N_INPUT), lambda i: (0, 0, 0)),
        ],
        out_specs=pl.BlockSpec((TA, N_INPUT), lambda i: (i, 0)),
        out_shape=jax.ShapeDtypeStruct((NPAD, N_INPUT), jnp.float32),
    )(xpad, Wr)

    seg = pl.kernel(
        _seg_sum_sc,
        out_type=jax.ShapeDtypeStruct((NCORE, BATCH, N_INPUT), jnp.float32),
        mesh=plsc.VectorSubcoreMesh(core_axis_name="c", subcore_axis_name="s"),
        scratch_types=[
            pltpu.VMEM((CH, N_INPUT), jnp.float32),
            pltpu.VMEM((KPW, 128), jnp.int32),
            pltpu.VMEM_SHARED((RACC, N_INPUT), jnp.float32),
        ],
    )
    partials = seg(y, ids3d, zeros)

    mol = pl.pallas_call(
        _finish_body,
        grid=(BATCH // TA,),
        in_specs=[
            pl.BlockSpec((NCORE, TA, N_INPUT), lambda i: (0, i, 0)),
            pl.BlockSpec((1, N_INPUT), lambda i: (0, 0)),
        ],
        out_specs=pl.BlockSpec((TA, N_INPUT), lambda i: (i, 0)),
        out_shape=jax.ShapeDtypeStruct((BATCH, N_INPUT), jnp.float32),
    )(partials, b.reshape(1, N_INPUT))
    return mol


# speedup vs baseline: 12.8591x; 1.3415x over previous
import functools

import jax
import jax.numpy as jnp
import numpy as np
from jax import lax
from jax.experimental import pallas as pl
from jax.experimental.pallas import tpu as pltpu
from jax.experimental.pallas import tpu_sc as plsc

N_ATOMS = 100000
N_INPUT = 128
BATCH = 4096
NK = 11

_MEMBERSHIPS = [(-1.645, 0.283), (-1.08, 0.17), (-0.739, 0.134),
                (-0.468, 0.118), (-0.228, 0.114), (0.0, 0.114),
                (0.228, 0.114), (0.468, 0.118), (0.739, 0.134),
                (1.08, 0.17), (1.645, 0.283)]
_MUS = [float(m) for m, _ in _MEMBERSHIPS]
_INV_SIG = [1.0 / float(s) for _, s in _MEMBERSHIPS]
_LOG2E = 1.4426950408889634
_A2 = [-0.5 * iv * iv * _LOG2E for iv in _INV_SIG]

NPAD = 102400
TA = 2048
TF = 1024

NCORE = 2
NSUB = 16
PER_W = NPAD // (NCORE * NSUB)
CH = 640
NCHUNK = PER_W // CH
KPC = CH // 128
KPW = PER_W // 128
ZPW = 264
RACC = ZPW * NSUB
OPW = BATCH // NSUB


def _hist_matmul_body(x_ref, w_ref, y_ref):
    x = x_ref[...]
    es = []
    s = None
    for k in range(NK):
        u = x - _MUS[k]
        e = jnp.exp2((u * u) * _A2[k])
        es.append(e)
        s = e if s is None else s + e
    r = 1.0 / s
    acc = None
    for k in range(NK):
        p = (es[k] * r).astype(jnp.bfloat16)
        t = jnp.dot(p, w_ref[k], preferred_element_type=jnp.float32)
        acc = t if acc is None else acc + t
    y_ref[...] = acc


def _seg_sum_sc(y_hbm, ids_hbm, zeros_hbm, out_hbm, rows_v, ids_v, acc_sh):
    c = lax.axis_index("c")
    s = lax.axis_index("s")
    w = c * NSUB + s
    pltpu.sync_copy(zeros_hbm.at[pl.ds(s * ZPW, ZPW)], rows_v.at[pl.ds(0, ZPW)])
    pltpu.sync_copy(rows_v.at[pl.ds(0, ZPW)], acc_sh.at[pl.ds(s * ZPW, ZPW)])
    pltpu.sync_copy(ids_hbm.at[w], ids_v)
    plsc.subcore_barrier()
    for t in range(NCHUNK):
        base = w * PER_W + t * CH
        pltpu.sync_copy(y_hbm.at[pl.ds(base, CH)], rows_v)
        for j in range(KPC):
            pltpu.sync_copy(rows_v.at[pl.ds(j * 128, 128)],
                            acc_sh.at[ids_v.at[t * KPC + j]], add=True)
    plsc.subcore_barrier()
    pltpu.sync_copy(acc_sh.at[pl.ds(s * OPW, OPW)], rows_v.at[pl.ds(0, OPW)])
    pltpu.sync_copy(rows_v.at[pl.ds(0, OPW)],
                    out_hbm.at[c, pl.ds(s * OPW, OPW)])


def _finish_body(p_ref, b_ref, o_ref):
    o_ref[...] = jnp.tanh(p_ref[0] + p_ref[1] + b_ref[...])


@jax.jit
def kernel(outputs, atom_split, W, b):
    Wr = W.reshape(N_INPUT, NK, N_INPUT).transpose(1, 0, 2).astype(jnp.bfloat16)
    ids = jnp.concatenate([
        atom_split.astype(jnp.int32),
        jnp.full((NPAD - N_ATOMS,), BATCH, jnp.int32),
    ])
    ids3d = ids.reshape(NCORE * NSUB, KPW, 128)
    zeros = jnp.zeros((RACC, N_INPUT), jnp.float32)

    last_blk = (N_ATOMS - 1) // TA
    y = pl.pallas_call(
        _hist_matmul_body,
        grid=(NPAD // TA,),
        in_specs=[
            pl.BlockSpec((TA, N_INPUT), lambda i: (jnp.minimum(i, last_blk), 0)),
            pl.BlockSpec((NK, N_INPUT, N_INPUT), lambda i: (0, 0, 0)),
        ],
        out_specs=pl.BlockSpec((TA, N_INPUT), lambda i: (i, 0)),
        out_shape=jax.ShapeDtypeStruct((NPAD, N_INPUT), jnp.float32),
    )(outputs, Wr)

    seg = pl.kernel(
        _seg_sum_sc,
        out_type=jax.ShapeDtypeStruct((NCORE, BATCH, N_INPUT), jnp.float32),
        mesh=plsc.VectorSubcoreMesh(core_axis_name="c", subcore_axis_name="s"),
        scratch_types=[
            pltpu.VMEM((CH, N_INPUT), jnp.float32),
            pltpu.VMEM((KPW, 128), jnp.int32),
            pltpu.VMEM_SHARED((RACC, N_INPUT), jnp.float32),
        ],
    )
    partials = seg(y, ids3d, zeros)

    mol = pl.pallas_call(
        _finish_body,
        grid=(BATCH // TF,),
        in_specs=[
            pl.BlockSpec((NCORE, TF, N_INPUT), lambda i: (0, i, 0)),
            pl.BlockSpec((1, N_INPUT), lambda i: (0, 0)),
        ],
        out_specs=pl.BlockSpec((TF, N_INPUT), lambda i: (i, 0)),
        out_shape=jax.ShapeDtypeStruct((BATCH, N_INPUT), jnp.float32),
    )(partials, b.reshape(1, N_INPUT))
    return mol


# speedup vs baseline: 13.2964x; 1.0340x over previous
import functools

import jax
import jax.numpy as jnp
import numpy as np
from jax import lax
from jax.experimental import pallas as pl
from jax.experimental.pallas import tpu as pltpu
from jax.experimental.pallas import tpu_sc as plsc

N_ATOMS = 100000
N_INPUT = 128
BATCH = 4096
NK = 11

_MEMBERSHIPS = [(-1.645, 0.283), (-1.08, 0.17), (-0.739, 0.134),
                (-0.468, 0.118), (-0.228, 0.114), (0.0, 0.114),
                (0.228, 0.114), (0.468, 0.118), (0.739, 0.134),
                (1.08, 0.17), (1.645, 0.283)]
_MUS = [float(m) for m, _ in _MEMBERSHIPS]
_INV_SIG = [1.0 / float(s) for _, s in _MEMBERSHIPS]
_LOG2E = 1.4426950408889634
_A2 = [-0.5 * iv * iv * _LOG2E for iv in _INV_SIG]

NHALF = 53248
NPAD = 2 * NHALF
TA = 2048
TF = 1024

NCORE = 2
NSUB = 16
PER_W = NHALF // (NCORE * NSUB)
CHUNKS = [(0, 640), (640, 640), (1280, 384)]
KPW = PER_W // 128
ZPW = 264
RACC = ZPW * NSUB
OPW = BATCH // NSUB


def _hist_matmul_body(x_ref, w_ref, y_ref):
    x = x_ref[...]
    es = []
    s = None
    for k in range(NK):
        u = x - _MUS[k]
        e = jnp.exp2((u * u) * _A2[k])
        es.append(e)
        s = e if s is None else s + e
    r = 1.0 / s
    acc = None
    for k in range(NK):
        p = (es[k] * r).astype(jnp.bfloat16)
        t = jnp.dot(p, w_ref[k], preferred_element_type=jnp.float32)
        acc = t if acc is None else acc + t
    y_ref[...] = acc


def _seg_sum_sc(y_hbm, ids_hbm, zeros_hbm, out_hbm, rows_v, ids_v, acc_sh):
    c = lax.axis_index("c")
    s = lax.axis_index("s")
    w = c * NSUB + s
    pltpu.sync_copy(zeros_hbm.at[pl.ds(s * ZPW, ZPW)], rows_v.at[pl.ds(0, ZPW)])
    pltpu.sync_copy(rows_v.at[pl.ds(0, ZPW)], acc_sh.at[pl.ds(s * ZPW, ZPW)])
    pltpu.sync_copy(ids_hbm.at[w], ids_v)
    plsc.subcore_barrier()
    for (start, ch) in CHUNKS:
        base = w * PER_W + start
        pltpu.sync_copy(y_hbm.at[pl.ds(base, ch)], rows_v.at[pl.ds(0, ch)])
        for j in range(ch // 128):
            pltpu.sync_copy(rows_v.at[pl.ds(j * 128, 128)],
                            acc_sh.at[ids_v.at[start // 128 + j]], add=True)
    plsc.subcore_barrier()
    pltpu.sync_copy(acc_sh.at[pl.ds(s * OPW, OPW)], rows_v.at[pl.ds(0, OPW)])
    pltpu.sync_copy(rows_v.at[pl.ds(0, OPW)],
                    out_hbm.at[c, pl.ds(s * OPW, OPW)])


def _finish_body(pa_ref, pb_ref, b_ref, o_ref):
    o_ref[...] = jnp.tanh(pa_ref[0] + pa_ref[1] + pb_ref[0] + pb_ref[1]
                          + b_ref[...])


@jax.jit
def kernel(outputs, atom_split, W, b):
    Wr = W.reshape(N_INPUT, NK, N_INPUT).transpose(1, 0, 2).astype(jnp.bfloat16)
    ids = jnp.concatenate([
        atom_split.astype(jnp.int32),
        jnp.full((NPAD - N_ATOMS,), BATCH, jnp.int32),
    ])
    zeros = jnp.zeros((RACC, N_INPUT), jnp.float32)

    last_blk = (N_ATOMS - 1) // TA
    blocks_per_half = NHALF // TA
    seg = pl.kernel(
        _seg_sum_sc,
        out_type=jax.ShapeDtypeStruct((NCORE, BATCH, N_INPUT), jnp.float32),
        mesh=plsc.VectorSubcoreMesh(core_axis_name="c", subcore_axis_name="s"),
        scratch_types=[
            pltpu.VMEM((640, N_INPUT), jnp.float32),
            pltpu.VMEM((KPW, 128), jnp.int32),
            pltpu.VMEM_SHARED((RACC, N_INPUT), jnp.float32),
        ],
    )
    partials = []
    for h in range(2):
        base_blk = h * blocks_per_half
        y_h = pl.pallas_call(
            _hist_matmul_body,
            grid=(blocks_per_half,),
            in_specs=[
                pl.BlockSpec(
                    (TA, N_INPUT),
                    lambda i, bb=base_blk: (jnp.minimum(bb + i, last_blk), 0)),
                pl.BlockSpec((NK, N_INPUT, N_INPUT), lambda i: (0, 0, 0)),
            ],
            out_specs=pl.BlockSpec((TA, N_INPUT), lambda i: (i, 0)),
            out_shape=jax.ShapeDtypeStruct((NHALF, N_INPUT), jnp.float32),
        )(outputs, Wr)
        ids3d_h = lax.dynamic_slice_in_dim(ids, h * NHALF, NHALF).reshape(
            NCORE * NSUB, KPW, 128)
        partials.append(seg(y_h, ids3d_h, zeros))

    mol = pl.pallas_call(
        _finish_body,
        grid=(BATCH // TF,),
        in_specs=[
            pl.BlockSpec((NCORE, TF, N_INPUT), lambda i: (0, i, 0)),
            pl.BlockSpec((NCORE, TF, N_INPUT), lambda i: (0, i, 0)),
            pl.BlockSpec((1, N_INPUT), lambda i: (0, 0)),
        ],
        out_specs=pl.BlockSpec((TF, N_INPUT), lambda i: (i, 0)),
        out_shape=jax.ShapeDtypeStruct((BATCH, N_INPUT), jnp.float32),
    )(partials[0], partials[1], b.reshape(1, N_INPUT))
    return mol


# speedup vs baseline: 14.1588x; 1.0649x over previous
import functools

import jax
import jax.numpy as jnp
import numpy as np
from jax import lax
from jax.experimental import pallas as pl
from jax.experimental.pallas import tpu as pltpu
from jax.experimental.pallas import tpu_sc as plsc

N_ATOMS = 100000
N_INPUT = 128
BATCH = 4096
NK = 11

_MEMBERSHIPS = [(-1.645, 0.283), (-1.08, 0.17), (-0.739, 0.134),
                (-0.468, 0.118), (-0.228, 0.114), (0.0, 0.114),
                (0.228, 0.114), (0.468, 0.118), (0.739, 0.134),
                (1.08, 0.17), (1.645, 0.283)]
_MUS = [float(m) for m, _ in _MEMBERSHIPS]
_INV_SIG = [1.0 / float(s) for _, s in _MEMBERSHIPS]
_LOG2E = 1.4426950408889634
_A2 = [-0.5 * iv * iv * _LOG2E for iv in _INV_SIG]
_D2 = [a * m * m for a, m in zip(_A2, _MUS)]
_G2 = [-2.0 * a * m for a, m in zip(_A2, _MUS)]

TA = 2048
TF = 1024
NPART = [81920, 24576]
NPAD = sum(NPART)

NCORE = 2
NSUB = 16
CHUNKS = [[(0, 640), (640, 640), (1280, 640), (1920, 640)],
          [(0, 640), (640, 128)]]
ZPW = 264
RACC = ZPW * NSUB
OPW = BATCH // NSUB


def _hist_matmul_body(x_ref, w_ref, y_ref):
    x = x_ref[...]
    q = x * x
    es = [None] * NK
    for k in range(5):
        t = q * _A2[k] + _D2[k]
        u = x * _G2[k]
        es[k] = jnp.exp2(t + u)
        es[10 - k] = jnp.exp2(t - u)
    es[5] = jnp.exp2(q * _A2[5])
    s = None
    for k in range(NK):
        s = es[k] if s is None else s + es[k]
    r = 1.0 / s
    acc = None
    for k in range(NK):
        p = (es[k] * r).astype(jnp.bfloat16)
        t = jnp.dot(p, w_ref[k], preferred_element_type=jnp.float32)
        acc = t if acc is None else acc + t
    y_ref[...] = acc


def _seg_sum_sc(y_hbm, ids_hbm, zeros_hbm, out_hbm, rows_v, ids_v, acc_sh,
                *, chunks, per_w):
    c = lax.axis_index("c")
    s = lax.axis_index("s")
    w = c * NSUB + s
    pltpu.sync_copy(zeros_hbm.at[pl.ds(s * ZPW, ZPW)], rows_v.at[pl.ds(0, ZPW)])
    pltpu.sync_copy(rows_v.at[pl.ds(0, ZPW)], acc_sh.at[pl.ds(s * ZPW, ZPW)])
    pltpu.sync_copy(ids_hbm.at[w], ids_v)
    plsc.subcore_barrier()
    for (start, ch) in chunks:
        base = w * per_w + start
        pltpu.sync_copy(y_hbm.at[pl.ds(base, ch)], rows_v.at[pl.ds(0, ch)])
        for j in range(ch // 128):
            pltpu.sync_copy(rows_v.at[pl.ds(j * 128, 128)],
                            acc_sh.at[ids_v.at[start // 128 + j]], add=True)
    plsc.subcore_barrier()
    pltpu.sync_copy(acc_sh.at[pl.ds(s * OPW, OPW)], rows_v.at[pl.ds(0, OPW)])
    pltpu.sync_copy(rows_v.at[pl.ds(0, OPW)],
                    out_hbm.at[c, pl.ds(s * OPW, OPW)])


def _finish_body(pa_ref, pb_ref, b_ref, o_ref):
    o_ref[...] = jnp.tanh(pa_ref[0] + pa_ref[1] + pb_ref[0] + pb_ref[1]
                          + b_ref[...])


@jax.jit
def kernel(outputs, atom_split, W, b):
    Wr = W.reshape(N_INPUT, NK, N_INPUT).transpose(1, 0, 2).astype(jnp.bfloat16)
    ids = jnp.concatenate([
        atom_split.astype(jnp.int32),
        jnp.full((NPAD - N_ATOMS,), BATCH, jnp.int32),
    ])
    zeros = jnp.zeros((RACC, N_INPUT), jnp.float32)

    last_blk = (N_ATOMS - 1) // TA
    partials = []
    row0 = 0
    for h in range(2):
        nrows = NPART[h]
        base_blk = row0 // TA
        y_h = pl.pallas_call(
            _hist_matmul_body,
            grid=(nrows // TA,),
            in_specs=[
                pl.BlockSpec(
                    (TA, N_INPUT),
                    lambda i, bb=base_blk: (jnp.minimum(bb + i, last_blk), 0)),
                pl.BlockSpec((NK, N_INPUT, N_INPUT), lambda i: (0, 0, 0)),
            ],
            out_specs=pl.BlockSpec((TA, N_INPUT), lambda i: (i, 0)),
            out_shape=jax.ShapeDtypeStruct((nrows, N_INPUT), jnp.float32),
        )(outputs, Wr)
        per_w = nrows // (NCORE * NSUB)
        seg = pl.kernel(
            functools.partial(_seg_sum_sc, chunks=CHUNKS[h], per_w=per_w),
            out_type=jax.ShapeDtypeStruct((NCORE, BATCH, N_INPUT), jnp.float32),
            mesh=plsc.VectorSubcoreMesh(core_axis_name="c", subcore_axis_name="s"),
            scratch_types=[
                pltpu.VMEM((640, N_INPUT), jnp.float32),
                pltpu.VMEM((per_w // 128, 128), jnp.int32),
                pltpu.VMEM_SHARED((RACC, N_INPUT), jnp.float32),
            ],
        )
        ids3d_h = lax.dynamic_slice_in_dim(ids, row0, nrows).reshape(
            NCORE * NSUB, per_w // 128, 128)
        partials.append(seg(y_h, ids3d_h, zeros))
        row0 += nrows

    mol = pl.pallas_call(
        _finish_body,
        grid=(BATCH // TF,),
        in_specs=[
            pl.BlockSpec((NCORE, TF, N_INPUT), lambda i: (0, i, 0)),
            pl.BlockSpec((NCORE, TF, N_INPUT), lambda i: (0, i, 0)),
            pl.BlockSpec((1, N_INPUT), lambda i: (0, 0)),
        ],
        out_specs=pl.BlockSpec((TF, N_INPUT), lambda i: (i, 0)),
        out_shape=jax.ShapeDtypeStruct((BATCH, N_INPUT), jnp.float32),
    )(partials[0], partials[1], b.reshape(1, N_INPUT))
    return mol
